# baseline (speedup 1.0000x reference)
import functools

import jax
import jax.numpy as jnp
from jax.experimental import pallas as pl
from jax.experimental.pallas import tpu as pltpu

VOCAB_ = 1000000
DIMS_ = 64
BLK = 32768
NB = (VOCAB_ + BLK - 1) // BLK
NEG_INF = float("-inf")


def _body(idx_ref, table_ref, w_ref, b_ref, out_ref, z_ref, m_ref, s_ref):
    t = pl.program_id(0)

    @pl.when(t == 0)
    def _init():
        m_ref[...] = jnp.full_like(m_ref, NEG_INF)
        s_ref[...] = jnp.zeros_like(s_ref)

    @pl.when(t < NB)
    def _compute():
        r = idx_ref[0] % 8
        e = table_ref[pl.ds(r, 1), :]
        zb = jax.lax.dot_general(
            e, w_ref[...], (((1,), (1,)), ((), ())),
            preferred_element_type=jnp.float32,
        )
        zb = zb + b_ref[...]
        z_ref[:, pl.ds(t * BLK, BLK)] = zb
        col = t * BLK + jax.lax.broadcasted_iota(jnp.int32, (1, BLK), 1)
        zm = jnp.where(col < VOCAB_, zb, NEG_INF)
        bm = jnp.max(zm, axis=1, keepdims=True)
        new_m = jnp.maximum(m_ref[...], bm)
        s_ref[...] = s_ref[...] * jnp.exp(m_ref[...] - new_m) + jnp.sum(
            jnp.exp(zm - new_m), axis=1, keepdims=True)
        m_ref[...] = new_m

    @pl.when(t >= NB)
    def _write():
        j = t - NB
        lse = m_ref[...] + jnp.log(s_ref[...])
        out_ref[...] = z_ref[:, pl.ds(j * BLK, BLK)] - lse


@jax.jit
def _run(inputs, table, W, b2d):
    grid_spec = pltpu.PrefetchScalarGridSpec(
        num_scalar_prefetch=1,
        grid=(2 * NB,),
        in_specs=[
            pl.BlockSpec((8, DIMS_), lambda t, idx: (idx[0] // 8, 0)),
            pl.BlockSpec((BLK, DIMS_), lambda t, idx: (jnp.minimum(t, NB - 1), 0)),
            pl.BlockSpec((1, BLK), lambda t, idx: (0, jnp.minimum(t, NB - 1))),
        ],
        out_specs=pl.BlockSpec(
            (1, BLK), lambda t, idx: (0, jnp.where(t < NB, 0, t - NB))),
        scratch_shapes=[
            pltpu.VMEM((1, NB * BLK), jnp.float32),
            pltpu.VMEM((1, 1), jnp.float32),
            pltpu.VMEM((1, 1), jnp.float32),
        ],
    )
    return pl.pallas_call(
        _body,
        grid_spec=grid_spec,
        out_shape=jax.ShapeDtypeStruct((1, VOCAB_), jnp.float32),
        compiler_params=pltpu.CompilerParams(
            dimension_semantics=("arbitrary",),
        ),
    )(inputs, table, W, b2d)


def kernel(inputs, table, W, b):
    idx = inputs.astype(jnp.int32)
    return _run(idx, table, W, b.reshape(1, VOCAB_))
